# baseline (speedup 1.0000x reference)
import functools

import jax
import jax.numpy as jnp
from jax import lax
from jax.experimental import pallas as pl
from jax.experimental.pallas import tpu as pltpu
from jax.experimental.pallas import tpu_sc as plsc

_VOCAB = 256
_N_LANGS = 100
_D = 64
_B = 4096
_S = 50

_info = plsc.get_sparse_core_info()
_NC, _NS = _info.num_cores, _info.num_subcores
_BT = _B // 128
_NPAIR = _S // 2


def _proj_body(se_ref, le_ref, w_ref, b_ref, sp_ref, lp_ref):
    w = w_ref[...]
    sp_ref[...] = lax.dot_general(se_ref[...], w[:, :_D],
                                  (((1,), (1,)), ((), ())),
                                  preferred_element_type=jnp.float32)
    lp_ref[...] = lax.dot_general(le_ref[...], w[:, _D:],
                                  (((1,), (1,)), ((), ())),
                                  preferred_element_type=jnp.float32) + b_ref[...]


def _build_proj(se, le, w, b2):
    return pl.pallas_call(
        _proj_body,
        out_shape=(
            jax.ShapeDtypeStruct((_VOCAB, _D), jnp.float32),
            jax.ShapeDtypeStruct((_N_LANGS, _D), jnp.float32),
        ),
    )(se, le, w, b2)


_mesh = plsc.VectorSubcoreMesh(core_axis_name="c", subcore_axis_name="s")


@functools.partial(
    pl.kernel,
    mesh=_mesh,
    compiler_params=pltpu.CompilerParams(use_tc_tiling_on_sc=False,
                                         needs_layout_passes=False),
    out_type=jax.ShapeDtypeStruct((_S, _D // 8, _BT, 8, 128), jnp.float32),
    scratch_types=[
        pltpu.VMEM((128, _S), jnp.int32),
        pltpu.VMEM((128,), jnp.int32),
        pltpu.VMEM((128, _D), jnp.float32),
        pltpu.VMEM((128,), jnp.int32),
        pltpu.VMEM((128,), jnp.int32),
        pltpu.VMEM((128, _D), jnp.float32),
        pltpu.VMEM((128, _D), jnp.float32),
        pltpu.VMEM((_D // 8, 8, 129), jnp.float32),
        pltpu.VMEM((_D // 8, 8, 129), jnp.float32),
        pltpu.SemaphoreType.DMA,
        pltpu.SemaphoreType.DMA,
        pltpu.SemaphoreType.DMA,
        pltpu.SemaphoreType.DMA,
    ],
)
def _sc_gather(sp_hbm, lp_hbm, char_hbm, lang_hbm, out_hbm,
               char_blk, lang_v, lpr, ilist_a, ilist_b,
               rows_a, rows_b, tiles_a, tiles_b,
               gsem_a, gsem_b, ssem_a, ssem_b):
    wid = lax.axis_index("s") * _NC + lax.axis_index("c")

    pltpu.sync_copy(char_hbm.at[pl.ds(wid * 128, 128)], char_blk)
    pltpu.sync_copy(lang_hbm.at[pl.ds(wid * 128, 128)], lang_v)
    pltpu.async_copy(lp_hbm.at[lang_v], lpr, gsem_a)
    pltpu.make_async_copy(lp_hbm.at[lang_v], lpr, gsem_a).wait()

    lane = lax.iota(jnp.int32, 16)
    zero16 = jnp.full((16,), 0, jnp.int32)
    toks = [lane + lg * 16 for lg in range(8)]
    dtvs = [lane // 8 + 2 * k for k in range(4)]
    drv = lane % 8

    def build_ilist(s, ilist):
        for lg in range(8):
            v = plsc.load_gather(char_blk, [toks[lg], zero16 + s])
            ilist[pl.ds(lg * 16, 16)] = v

    def fire_gather(ilist, rows, gsem):
        pltpu.async_copy(sp_hbm.at[ilist], rows, gsem)

    def drain_gather(ilist, rows, gsem):
        pltpu.make_async_copy(sp_hbm.at[ilist], rows, gsem).wait()

    def transpose_add(rows, tiles):
        @plsc.parallel_loop(0, 128, unroll=4)
        def tbody(tok):
            bcv = zero16 + tok
            for k in range(4):
                v = rows[tok, pl.ds(k * 16, 16)] + lpr[tok, pl.ds(k * 16, 16)]
                plsc.store_scatter(tiles, [dtvs[k], drv, bcv], v)

    def fire_stores(s, tiles, ssem):
        for dt in range(_D // 8):
            pltpu.async_copy(tiles.at[dt, :, pl.ds(0, 128)],
                             out_hbm.at[s, dt, wid], ssem)

    def wait_stores(tiles, ssem):
        for dt in range(_D // 8):
            pltpu.make_async_copy(tiles.at[dt, :, pl.ds(0, 128)],
                                  out_hbm.at[0, dt, wid], ssem).wait()

    build_ilist(0, ilist_a)
    fire_gather(ilist_a, rows_a, gsem_a)

    def body(j, _):
        sa = 2 * j
        sb = 2 * j + 1

        @pl.when(j > 0)
        def _():
            wait_stores(tiles_b, ssem_b)
        build_ilist(sb, ilist_b)
        fire_gather(ilist_b, rows_b, gsem_b)

        @pl.when(j > 0)
        def _():
            wait_stores(tiles_a, ssem_a)
        drain_gather(ilist_a, rows_a, gsem_a)
        transpose_add(rows_a, tiles_a)
        fire_stores(sa, tiles_a, ssem_a)

        @pl.when(j < _NPAIR - 1)
        def _():
            build_ilist(sa + 2, ilist_a)
            fire_gather(ilist_a, rows_a, gsem_a)
        drain_gather(ilist_b, rows_b, gsem_b)
        transpose_add(rows_b, tiles_b)
        fire_stores(sb, tiles_b, ssem_b)
        return 0

    lax.fori_loop(0, _NPAIR, body, 0)
    wait_stores(tiles_a, ssem_a)
    wait_stores(tiles_b, ssem_b)


def kernel(char, lang, source_embedding, lang_embedding, fc_w, fc_b):
    sp, lp = _build_proj(source_embedding, lang_embedding, fc_w, fc_b[None, :])
    xt = _sc_gather(sp, lp, char, lang)
    return xt.transpose(2, 4, 0, 1, 3).reshape(_B, _S, _D)


# speedup vs baseline: 1.9352x; 1.9352x over previous
import functools

import jax
import jax.numpy as jnp
from jax import lax
from jax.experimental import pallas as pl
from jax.experimental.pallas import tpu as pltpu
from jax.experimental.pallas import tpu_sc as plsc

_VOCAB = 256
_N_LANGS = 100
_D = 64
_B = 4096
_S = 50
_NTOK = _B * _S

_info = plsc.get_sparse_core_info()
_NC, _NS = _info.num_cores, _info.num_subcores
_NW = _NC * _NS
_BT = _B // 128
_NPAIR = _S // 2


def _tables_body(char_ref, lang_ref, se_ref, le_ref, w_ref, b_ref,
                 table_ref, idx_ref):
    se = se_ref[...]
    le = le_ref[...]
    w = w_ref[...]
    b = b_ref[...]
    sp = lax.dot_general(se, w[:, :_D], (((1,), (1,)), ((), ())),
                         preferred_element_type=jnp.float32)
    lp = lax.dot_general(le, w[:, _D:], (((1,), (1,)), ((), ())),
                         preferred_element_type=jnp.float32) + b
    table_ref[...] = sp[:, None, :] + lp[None, :, :]
    idx_ref[...] = char_ref[...] * _N_LANGS + lang_ref[...]


def _build_tables(char, lang2, se, le, w, b2):
    return pl.pallas_call(
        _tables_body,
        out_shape=(
            jax.ShapeDtypeStruct((_VOCAB, _N_LANGS, _D), jnp.float32),
            jax.ShapeDtypeStruct((_B, _S), jnp.int32),
        ),
    )(char, lang2, se, le, w, b2)


_mesh = plsc.VectorSubcoreMesh(core_axis_name="c", subcore_axis_name="s")


@functools.partial(
    pl.kernel,
    mesh=_mesh,
    compiler_params=pltpu.CompilerParams(use_tc_tiling_on_sc=False,
                                         needs_layout_passes=False),
    out_type=jax.ShapeDtypeStruct((_S, _D // 8, _BT, 8, 128), jnp.float32),
    scratch_types=[
        pltpu.VMEM((128, _S), jnp.int32),
        pltpu.VMEM((5, 128), jnp.int32),
        pltpu.VMEM((5, 128, _D), jnp.float32),
        pltpu.VMEM((5, _D // 8, 8, 129), jnp.float32),
        pltpu.SemaphoreType.DMA,
        pltpu.SemaphoreType.DMA,
        pltpu.SemaphoreType.DMA,
        pltpu.SemaphoreType.DMA,
        pltpu.SemaphoreType.DMA,
        pltpu.SemaphoreType.DMA,
        pltpu.SemaphoreType.DMA,
        pltpu.SemaphoreType.DMA,
        pltpu.SemaphoreType.DMA,
        pltpu.SemaphoreType.DMA,
    ],
)
def _sc_gather(table_hbm, idx_hbm, out_hbm, idx_blk, ilists, rowss, tiless,
               g0, g1, g2, g3, g4, s0, s1, s2, s3, s4):
    gsems = [g0, g1, g2, g3, g4]
    ssems = [s0, s1, s2, s3, s4]
    wid = lax.axis_index("s") * _NC + lax.axis_index("c")

    pltpu.sync_copy(idx_hbm.at[pl.ds(wid * 128, 128)], idx_blk)

    lane = lax.iota(jnp.int32, 16)

    def build_ilist(s, ilist):
        for lg in range(8):
            v = plsc.load_gather(idx_blk, [lane + lg * 16,
                                           jnp.full((16,), 0, jnp.int32) + s])
            ilist[pl.ds(lg * 16, 16)] = v

    def fire_gather(ilist, rows, gsem):
        pltpu.async_copy(table_hbm.at[ilist], rows, gsem)

    def drain_gather(ilist, rows, gsem):
        pltpu.make_async_copy(table_hbm.at[ilist], rows, gsem).wait()

    zero16 = jnp.full((16,), 0, jnp.int32)
    dtvs = [lane // 8 + 2 * k for k in range(4)]
    drv = lane % 8

    def transpose(rows, tiles):
        @plsc.parallel_loop(0, 128, unroll=4)
        def tbody(tok):
            bcv = zero16 + tok
            for k in range(4):
                v = rows[tok, pl.ds(k * 16, 16)]
                plsc.store_scatter(tiles, [dtvs[k], drv, bcv], v)

    def fire_stores(s, tiles, ssem):
        for dt in range(_D // 8):
            pltpu.async_copy(tiles.at[dt, :, pl.ds(0, 128)],
                             out_hbm.at[s, dt, wid], ssem)

    def wait_stores(tiles, ssem):
        for dt in range(_D // 8):
            pltpu.make_async_copy(tiles.at[dt, :, pl.ds(0, 128)],
                                  out_hbm.at[0, dt, wid], ssem).wait()

    for q in range(5):
        build_ilist(q, ilists.at[q])
        fire_gather(ilists.at[q], rowss.at[q], gsems[q])

    def body(j, _):
        for q in range(5):
            sq = 5 * j + q

            @pl.when(j > 0)
            def _():
                wait_stores(tiless.at[q], ssems[q])
            drain_gather(ilists.at[q], rowss.at[q], gsems[q])
            transpose(rowss.at[q], tiless.at[q])
            fire_stores(sq, tiless.at[q], ssems[q])

            @pl.when(j < 9)
            def _():
                build_ilist(sq + 5, ilists.at[q])
                fire_gather(ilists.at[q], rowss.at[q], gsems[q])
        return 0

    lax.fori_loop(0, 10, body, 0)
    for q in range(5):
        wait_stores(tiless.at[q], ssems[q])


def kernel(char, lang, source_embedding, lang_embedding, fc_w, fc_b):
    table3, idx = _build_tables(char, lang[:, None], source_embedding,
                                lang_embedding, fc_w, fc_b[None, :])
    table = table3.reshape(_VOCAB * _N_LANGS, _D)
    xt = _sc_gather(table, idx)
    out = xt.transpose(2, 4, 0, 1, 3).reshape(_B, _S, _D)
    return out
